# baseline (speedup 1.0000x reference)
import functools

import jax
import jax.numpy as jnp
from jax import lax
from jax.experimental import pallas as pl
from jax.experimental.pallas import tpu as pltpu
from jax.experimental.pallas import tpu_sc as plsc

B = 64
T = 577
N = T - 1
K = N // 2
D = 768
R = B * (K + 1)
CHUNK = 64
NCHUNKS = R // CHUNK
NW = 32


def _gather_indices() -> jnp.ndarray:
    rand = jax.random.normal(jax.random.key(1), (B, N), dtype=jnp.float32)
    _, keep = lax.top_k(rand, K)
    rows_b = jnp.arange(B, dtype=jnp.int32)[:, None] * T
    kept = rows_b + 1 + keep.astype(jnp.int32)
    cls = rows_b
    g = jnp.concatenate([cls, kept], axis=1)
    return g.reshape(NCHUNKS, CHUNK)


@functools.cache
def _sc_gather():
    @functools.partial(
        pl.kernel,
        out_type=jax.ShapeDtypeStruct((R, D), jnp.float32),
        mesh=plsc.VectorSubcoreMesh(core_axis_name="c", subcore_axis_name="s"),
        scratch_types=[
            pltpu.VMEM((CHUNK,), jnp.int32),
            pltpu.VMEM((CHUNK, D), jnp.float32),
            pltpu.SemaphoreType.DMA,
        ],
    )
    def body(x_hbm, idx_hbm, out_hbm, idx_v, rows_v, sem):
        wid = lax.axis_index("s") * 2 + lax.axis_index("c")
        for i in range(10):
            t = wid + i * NW

            @pl.when(t < NCHUNKS)
            def _():
                pltpu.sync_copy(idx_hbm.at[t], idx_v)
                pltpu.async_copy(x_hbm.at[idx_v], rows_v, sem).wait()
                pltpu.sync_copy(rows_v, out_hbm.at[pl.ds(t * CHUNK, CHUNK)])

    return body


def kernel(x: jnp.ndarray) -> jnp.ndarray:
    xflat = x.reshape(B * T, D)
    out = _sc_gather()(xflat, _gather_indices())
    return out.reshape(B, K + 1, D)


# speedup vs baseline: 1.0277x; 1.0277x over previous
import functools

import jax
import jax.numpy as jnp
from jax import lax
from jax.experimental import pallas as pl
from jax.experimental.pallas import tpu as pltpu
from jax.experimental.pallas import tpu_sc as plsc

B = 64
T = 577
N = T - 1
K = N // 2
D = 768
R = B * (K + 1)
CHUNK = 64
NCHUNKS = R // CHUNK
NW = 32
MAXC = -(-NCHUNKS // NW)


def _gather_indices() -> jnp.ndarray:
    rand = jax.random.normal(jax.random.key(1), (B, N), dtype=jnp.float32)
    _, keep = lax.top_k(rand, K)
    rows_b = jnp.arange(B, dtype=jnp.int32)[:, None] * T
    kept = rows_b + 1 + keep.astype(jnp.int32)
    cls = rows_b
    g = jnp.concatenate([cls, kept], axis=1)
    chunks = g.reshape(NCHUNKS, CHUNK)
    pad = jnp.zeros((NW * MAXC - NCHUNKS, CHUNK), jnp.int32)
    chunks = jnp.concatenate([chunks, pad], axis=0)
    return chunks.reshape(MAXC, NW, CHUNK).transpose(1, 0, 2)


@functools.cache
def _sc_gather():
    @functools.partial(
        pl.kernel,
        out_type=jax.ShapeDtypeStruct((R, D), jnp.float32),
        mesh=plsc.VectorSubcoreMesh(core_axis_name="c", subcore_axis_name="s"),
        scratch_types=[
            pltpu.VMEM((MAXC, CHUNK), jnp.int32),
            pltpu.VMEM((CHUNK, D), jnp.float32),
            pltpu.VMEM((CHUNK, D), jnp.float32),
            pltpu.SemaphoreType.DMA,
            pltpu.SemaphoreType.DMA,
            pltpu.SemaphoreType.DMA,
            pltpu.SemaphoreType.DMA,
        ],
    )
    def body(x_hbm, idx_hbm, out_hbm, idx_v, buf0, buf1, sg0, sg1, ss0, ss1):
        wid = lax.axis_index("s") * 2 + lax.axis_index("c")
        bufs = (buf0, buf1)
        sgs = (sg0, sg1)
        sss = (ss0, ss1)

        pltpu.sync_copy(idx_hbm.at[wid], idx_v)

        def chunk_id(i):
            return wid + i * NW

        for i in range(2):
            @pl.when(chunk_id(i) < NCHUNKS)
            def _(i=i):
                pltpu.async_copy(x_hbm.at[idx_v.at[i]], bufs[i], sgs[i])

        for i in range(MAXC):
            b = i % 2
            t = chunk_id(i)

            @pl.when(t < NCHUNKS)
            def _(i=i, b=b, t=t):
                pltpu.make_async_copy(
                    x_hbm.at[idx_v.at[i]], bufs[b], sgs[b]).wait()
                pltpu.async_copy(
                    bufs[b], out_hbm.at[pl.ds(t * CHUNK, CHUNK)], sss[b])
                pltpu.make_async_copy(
                    bufs[b], out_hbm.at[pl.ds(t * CHUNK, CHUNK)], sss[b]).wait()

            if i + 2 < MAXC:
                @pl.when(chunk_id(i + 2) < NCHUNKS)
                def _(i=i, b=b):
                    pltpu.async_copy(
                        x_hbm.at[idx_v.at[i + 2]], bufs[b], sgs[b])

    return body


def kernel(x: jnp.ndarray) -> jnp.ndarray:
    xflat = x.reshape(B * T, D)
    out = _sc_gather()(xflat, _gather_indices())
    return out.reshape(B, K + 1, D)


# speedup vs baseline: 1.0319x; 1.0041x over previous
import functools

import jax
import jax.numpy as jnp
from jax import lax
from jax.experimental import pallas as pl
from jax.experimental.pallas import tpu as pltpu
from jax.experimental.pallas import tpu_sc as plsc

B = 64
T = 577
N = T - 1
K = N // 2
D = 768
R = B * (K + 1)
CHUNK = 64
NCHUNKS = R // CHUNK
NW = 32
MAXC = -(-NCHUNKS // NW)


def _gather_indices() -> jnp.ndarray:
    rand = jax.random.normal(jax.random.key(1), (B, N), dtype=jnp.float32)
    _, keep = lax.top_k(rand, K)
    rows_b = jnp.arange(B, dtype=jnp.int32)[:, None] * T
    kept = rows_b + 1 + keep.astype(jnp.int32)
    cls = rows_b
    g = jnp.concatenate([cls, kept], axis=1)
    chunks = g.reshape(NCHUNKS, CHUNK)
    pad = jnp.zeros((NW * MAXC - NCHUNKS, CHUNK), jnp.int32)
    chunks = jnp.concatenate([chunks, pad], axis=0)
    return chunks.reshape(MAXC, NW, CHUNK).transpose(1, 0, 2)


@functools.cache
def _sc_gather():
    @functools.partial(
        pl.kernel,
        out_type=jax.ShapeDtypeStruct((R, D), jnp.float32),
        mesh=plsc.VectorSubcoreMesh(core_axis_name="c", subcore_axis_name="s"),
        compiler_params=pltpu.CompilerParams(use_tc_tiling_on_sc=True),
        scratch_types=[
            pltpu.VMEM((MAXC, CHUNK), jnp.int32),
            pltpu.VMEM((CHUNK, D), jnp.float32),
            pltpu.VMEM((CHUNK, D), jnp.float32),
            pltpu.SemaphoreType.DMA,
            pltpu.SemaphoreType.DMA,
            pltpu.SemaphoreType.DMA,
            pltpu.SemaphoreType.DMA,
        ],
    )
    def body(x_hbm, idx_hbm, out_hbm, idx_v, buf0, buf1, sg0, sg1, ss0, ss1):
        wid = lax.axis_index("s") * 2 + lax.axis_index("c")
        bufs = (buf0, buf1)
        sgs = (sg0, sg1)
        sss = (ss0, ss1)

        pltpu.sync_copy(idx_hbm.at[wid], idx_v)

        def chunk_id(i):
            return wid + i * NW

        for i in range(2):
            @pl.when(chunk_id(i) < NCHUNKS)
            def _(i=i):
                pltpu.async_copy(x_hbm.at[idx_v.at[i]], bufs[i], sgs[i])

        for i in range(MAXC):
            b = i % 2
            t = chunk_id(i)

            @pl.when(t < NCHUNKS)
            def _(i=i, b=b, t=t):
                pltpu.make_async_copy(
                    x_hbm.at[idx_v.at[i]], bufs[b], sgs[b]).wait()
                pltpu.async_copy(
                    bufs[b], out_hbm.at[pl.ds(t * CHUNK, CHUNK)], sss[b])
                pltpu.make_async_copy(
                    bufs[b], out_hbm.at[pl.ds(t * CHUNK, CHUNK)], sss[b]).wait()

            if i + 2 < MAXC:
                @pl.when(chunk_id(i + 2) < NCHUNKS)
                def _(i=i, b=b):
                    pltpu.async_copy(
                        x_hbm.at[idx_v.at[i + 2]], bufs[b], sgs[b])

    return body


def kernel(x: jnp.ndarray) -> jnp.ndarray:
    xflat = x.reshape(B * T, D)
    out = _sc_gather()(xflat, _gather_indices())
    return out.reshape(B, K + 1, D)


# speedup vs baseline: 5.9487x; 5.7646x over previous
import functools

import jax
import jax.numpy as jnp
import numpy as np
from jax import lax
from jax.experimental import pallas as pl
from jax.experimental.pallas import tpu as pltpu
from jax.experimental.pallas import tpu_sc as plsc

B = 64
T = 577
N = T - 1
K = N // 2
D = 768
R = B * (K + 1)
CHUNK = B
NCHUNKS = K + 1
NW = 32
MAXC = -(-NCHUNKS // NW)


@functools.cache
def _gather_indices() -> np.ndarray:
    with jax.ensure_compile_time_eval():
        rand = jax.random.normal(jax.random.key(1), (B, N), dtype=jnp.float32)
        _, keep = lax.top_k(rand, K)
        tok = np.concatenate(
            [np.zeros((B, 1), np.int32), 1 + np.asarray(keep, np.int32)],
            axis=1)
    b = np.arange(B, dtype=np.int32)[None, :]
    q = tok.T * B + b
    pad = np.zeros((NW * MAXC - NCHUNKS, CHUNK), np.int32)
    q = np.concatenate([q, pad], axis=0)
    return np.ascontiguousarray(
        q.reshape(MAXC, NW, CHUNK).transpose(1, 0, 2))


@functools.cache
def _sc_gather():
    @functools.partial(
        pl.kernel,
        out_type=jax.ShapeDtypeStruct((R, D), jnp.float32),
        mesh=plsc.VectorSubcoreMesh(core_axis_name="c", subcore_axis_name="s"),
        compiler_params=pltpu.CompilerParams(use_tc_tiling_on_sc=True),
        scratch_types=[
            pltpu.VMEM((MAXC, CHUNK), jnp.int32),
            pltpu.VMEM((CHUNK, D), jnp.float32),
            pltpu.VMEM((CHUNK, D), jnp.float32),
            pltpu.SemaphoreType.DMA,
            pltpu.SemaphoreType.DMA,
            pltpu.SemaphoreType.DMA,
            pltpu.SemaphoreType.DMA,
        ],
    )
    def body(x_hbm, idx_hbm, out_hbm, idx_v, buf0, buf1, sg0, sg1, ss0, ss1):
        wid = lax.axis_index("s") * 2 + lax.axis_index("c")
        bufs = (buf0, buf1)
        sgs = (sg0, sg1)
        sss = (ss0, ss1)

        pltpu.sync_copy(idx_hbm.at[wid], idx_v)

        def chunk_id(i):
            return wid + i * NW

        for i in range(2):
            @pl.when(chunk_id(i) < NCHUNKS)
            def _(i=i):
                pltpu.async_copy(x_hbm.at[idx_v.at[i]], bufs[i], sgs[i])

        for i in range(MAXC):
            b = i % 2
            t = chunk_id(i)

            @pl.when(t < NCHUNKS)
            def _(i=i, b=b, t=t):
                pltpu.make_async_copy(
                    x_hbm.at[idx_v.at[i]], bufs[b], sgs[b]).wait()
                pltpu.async_copy(
                    bufs[b], out_hbm.at[pl.ds(t * CHUNK, CHUNK)], sss[b])
                pltpu.make_async_copy(
                    bufs[b], out_hbm.at[pl.ds(t * CHUNK, CHUNK)], sss[b]).wait()

            if i + 2 < MAXC:
                @pl.when(chunk_id(i + 2) < NCHUNKS)
                def _(i=i, b=b):
                    pltpu.async_copy(
                        x_hbm.at[idx_v.at[i + 2]], bufs[b], sgs[b])

    return body


def kernel(x: jnp.ndarray) -> jnp.ndarray:
    xflat = x.transpose(1, 0, 2).reshape(T * B, D)
    idx = jnp.asarray(_gather_indices())
    out = _sc_gather()(xflat, idx)
    return out.reshape(NCHUNKS, B, D).transpose(1, 0, 2)


# speedup vs baseline: 5.9584x; 1.0016x over previous
import functools

import jax
import jax.numpy as jnp
import numpy as np
from jax import lax
from jax.experimental import pallas as pl
from jax.experimental.pallas import tpu as pltpu
from jax.experimental.pallas import tpu_sc as plsc

B = 64
T = 577
N = T - 1
K = N // 2
D = 768
R = B * (K + 1)
CHUNK = 32
NCHUNKS = R // CHUNK
NW = 32
MAXC = -(-NCHUNKS // NW)
NBUF = 4


@functools.cache
def _gather_indices() -> np.ndarray:
    with jax.ensure_compile_time_eval():
        rand = jax.random.normal(jax.random.key(1), (B, N), dtype=jnp.float32)
        _, keep = lax.top_k(rand, K)
        tok = np.concatenate(
            [np.zeros((B, 1), np.int32), 1 + np.asarray(keep, np.int32)],
            axis=1)
    b = np.arange(B, dtype=np.int32)[None, :]
    q = tok.T * B + b
    q = q.reshape(NCHUNKS, CHUNK)
    pad = np.zeros((NW * MAXC - NCHUNKS, CHUNK), np.int32)
    q = np.concatenate([q, pad], axis=0)
    return np.ascontiguousarray(
        q.reshape(MAXC, NW, CHUNK).transpose(1, 0, 2))


@functools.cache
def _sc_gather():
    @functools.partial(
        pl.kernel,
        out_type=jax.ShapeDtypeStruct((R, D), jnp.float32),
        mesh=plsc.VectorSubcoreMesh(core_axis_name="c", subcore_axis_name="s"),
        compiler_params=pltpu.CompilerParams(use_tc_tiling_on_sc=True),
        scratch_types=[
            pltpu.VMEM((MAXC, CHUNK), jnp.int32),
            [pltpu.VMEM((CHUNK, D), jnp.float32) for _ in range(NBUF)],
            [pltpu.SemaphoreType.DMA for _ in range(NBUF)],
            [pltpu.SemaphoreType.DMA for _ in range(NBUF)],
        ],
    )
    def body(x_hbm, idx_hbm, out_hbm, idx_v, bufs, sgs, sss):
        wid = lax.axis_index("s") * 2 + lax.axis_index("c")

        pltpu.sync_copy(idx_hbm.at[wid], idx_v)

        def chunk_id(i):
            return wid + i * NW

        def out_slab(i):
            return out_hbm.at[pl.ds(chunk_id(i) * CHUNK, CHUNK)]

        for i in range(2):
            @pl.when(chunk_id(i) < NCHUNKS)
            def _(i=i):
                pltpu.async_copy(x_hbm.at[idx_v.at[i]], bufs[i], sgs[i])

        for i in range(MAXC):
            b = i % NBUF

            @pl.when(chunk_id(i) < NCHUNKS)
            def _(i=i, b=b):
                pltpu.make_async_copy(
                    x_hbm.at[idx_v.at[i]], bufs[b], sgs[b]).wait()
                pltpu.async_copy(bufs[b], out_slab(i), sss[b])

            if i + 2 < MAXC:
                b2 = (i + 2) % NBUF

                @pl.when(chunk_id(i + 2) < NCHUNKS)
                def _(i=i, b2=b2):
                    if i >= 2:
                        pltpu.make_async_copy(
                            bufs[b2], out_slab(i - 2), sss[b2]).wait()
                    pltpu.async_copy(
                        x_hbm.at[idx_v.at[i + 2]], bufs[b2], sgs[b2])

        for i in range(MAXC):
            cond = chunk_id(i) < NCHUNKS
            if i + 4 < MAXC:
                cond &= chunk_id(i + 4) >= NCHUNKS

            @pl.when(cond)
            def _(i=i):
                pltpu.make_async_copy(
                    bufs[i % NBUF], out_slab(i), sss[i % NBUF]).wait()

    return body


def kernel(x: jnp.ndarray) -> jnp.ndarray:
    xflat = x.transpose(1, 0, 2).reshape(T * B, D)
    idx = jnp.asarray(_gather_indices())
    out = _sc_gather()(xflat, idx)
    return out.reshape(K + 1, B, D).transpose(1, 0, 2)


# speedup vs baseline: 5.9726x; 1.0024x over previous
import functools

import jax
import jax.numpy as jnp
import numpy as np
from jax import lax
from jax.experimental import pallas as pl
from jax.experimental.pallas import tpu as pltpu
from jax.experimental.pallas import tpu_sc as plsc

B = 64
T = 577
N = T - 1
K = N // 2
D = 768
R = B * (K + 1)
CHUNK = 32
NCHUNKS = R // CHUNK
NW = 32
MAXC = -(-NCHUNKS // NW)
NBUF = 4
IDXW = -(-MAXC * CHUNK // 128) * 128


@functools.cache
def _gather_indices() -> np.ndarray:
    with jax.ensure_compile_time_eval():
        rand = jax.random.normal(jax.random.key(1), (B, N), dtype=jnp.float32)
        _, keep = lax.top_k(rand, K)
        tok = np.concatenate(
            [np.zeros((B, 1), np.int32), 1 + np.asarray(keep, np.int32)],
            axis=1)
    b = np.arange(B, dtype=np.int32)[None, :]
    q = tok.T * B + b
    q = q.reshape(NCHUNKS, CHUNK)
    pad = np.zeros((NW * MAXC - NCHUNKS, CHUNK), np.int32)
    q = np.concatenate([q, pad], axis=0)
    q = q.reshape(MAXC, NW, CHUNK).transpose(1, 0, 2).reshape(NW, MAXC * CHUNK)
    out = np.zeros((NW, IDXW), np.int32)
    out[:, :MAXC * CHUNK] = q
    return out


@functools.cache
def _sc_gather():
    @functools.partial(
        pl.kernel,
        out_type=jax.ShapeDtypeStruct((R, D), jnp.float32),
        mesh=plsc.VectorSubcoreMesh(core_axis_name="c", subcore_axis_name="s"),
        compiler_params=pltpu.CompilerParams(use_tc_tiling_on_sc=True),
        scratch_types=[
            pltpu.VMEM((IDXW,), jnp.int32),
            [pltpu.VMEM((CHUNK, D), jnp.float32) for _ in range(NBUF)],
            [pltpu.SemaphoreType.DMA for _ in range(NBUF)],
            [pltpu.SemaphoreType.DMA for _ in range(NBUF)],
        ],
    )
    def body(x_hbm, idx_hbm, out_hbm, idx_v, bufs, sgs, sss):
        wid = lax.axis_index("s") * 2 + lax.axis_index("c")

        pltpu.sync_copy(idx_hbm.at[wid], idx_v)

        def chunk_id(i):
            return wid + i * NW

        def out_slab(i):
            return out_hbm.at[pl.ds(chunk_id(i) * CHUNK, CHUNK)]

        for i in range(2):
            @pl.when(chunk_id(i) < NCHUNKS)
            def _(i=i):
                pltpu.async_copy(
                    x_hbm.at[idx_v.at[pl.ds(i * CHUNK, CHUNK)]],
                    bufs[i], sgs[i])

        for i in range(MAXC):
            b = i % NBUF

            @pl.when(chunk_id(i) < NCHUNKS)
            def _(i=i, b=b):
                pltpu.make_async_copy(
                    x_hbm.at[idx_v.at[pl.ds(i * CHUNK, CHUNK)]], bufs[b], sgs[b]).wait()
                pltpu.async_copy(bufs[b], out_slab(i), sss[b])

            if i + 2 < MAXC:
                b2 = (i + 2) % NBUF

                @pl.when(chunk_id(i + 2) < NCHUNKS)
                def _(i=i, b2=b2):
                    if i >= 2:
                        pltpu.make_async_copy(
                            bufs[b2], out_slab(i - 2), sss[b2]).wait()
                    pltpu.async_copy(
                        x_hbm.at[idx_v.at[pl.ds((i + 2) * CHUNK, CHUNK)]],
                        bufs[b2], sgs[b2])

        for i in range(MAXC):
            cond = chunk_id(i) < NCHUNKS
            if i + 4 < MAXC:
                cond &= chunk_id(i + 4) >= NCHUNKS

            @pl.when(cond)
            def _(i=i):
                pltpu.make_async_copy(
                    bufs[i % NBUF], out_slab(i), sss[i % NBUF]).wait()

    return body


def kernel(x: jnp.ndarray) -> jnp.ndarray:
    xflat = x.transpose(1, 0, 2).reshape(T * B, D)
    idx = jnp.asarray(_gather_indices())
    out = _sc_gather()(xflat, idx)
    return out.reshape(K + 1, B, D).transpose(1, 0, 2)


# speedup vs baseline: 5.9734x; 1.0001x over previous
import functools

import jax
import jax.numpy as jnp
import numpy as np
from jax import lax
from jax.experimental import pallas as pl
from jax.experimental.pallas import tpu as pltpu
from jax.experimental.pallas import tpu_sc as plsc

B = 64
T = 577
N = T - 1
K = N // 2
D = 768
R = B * (K + 1)
CHUNK = 32
NCHUNKS = R // CHUNK
NW = 32
MAXC = -(-NCHUNKS // NW)
NBUF = 4
IDXW = -(-MAXC * CHUNK // 128) * 128


@functools.cache
def _gather_indices() -> np.ndarray:
    with jax.ensure_compile_time_eval():
        rand = jax.random.normal(jax.random.key(1), (B, N), dtype=jnp.float32)
        _, keep = lax.top_k(rand, K)
        tok = np.concatenate(
            [np.zeros((B, 1), np.int32), 1 + np.asarray(keep, np.int32)],
            axis=1)
    b = np.arange(B, dtype=np.int32)[None, :]
    q = tok.T * B + b
    q = q.reshape(NCHUNKS, CHUNK)
    pad = np.zeros((NW * MAXC - NCHUNKS, CHUNK), np.int32)
    q = np.concatenate([q, pad], axis=0)
    q = q.reshape(MAXC, NW, CHUNK).transpose(1, 0, 2).reshape(NW, MAXC * CHUNK)
    out = np.zeros((NW, IDXW), np.int32)
    out[:, :MAXC * CHUNK] = q
    return out


@functools.cache
def _sc_gather():
    @functools.partial(
        pl.kernel,
        out_type=jax.ShapeDtypeStruct((R, D), jnp.float32),
        mesh=plsc.VectorSubcoreMesh(core_axis_name="c", subcore_axis_name="s"),
        compiler_params=pltpu.CompilerParams(
            use_tc_tiling_on_sc=True, skip_device_barrier=True),
        scratch_types=[
            pltpu.VMEM((IDXW,), jnp.int32),
            [pltpu.VMEM((CHUNK, D), jnp.float32) for _ in range(NBUF)],
            [pltpu.SemaphoreType.DMA for _ in range(NBUF)],
            [pltpu.SemaphoreType.DMA for _ in range(NBUF)],
        ],
    )
    def body(x_hbm, idx_hbm, out_hbm, idx_v, bufs, sgs, sss):
        wid = lax.axis_index("s") * 2 + lax.axis_index("c")

        pltpu.sync_copy(idx_hbm.at[wid], idx_v)

        def chunk_id(i):
            return wid + i * NW

        def out_slab(i):
            return out_hbm.at[pl.ds(chunk_id(i) * CHUNK, CHUNK)]

        for i in range(2):
            @pl.when(chunk_id(i) < NCHUNKS)
            def _(i=i):
                pltpu.async_copy(
                    x_hbm.at[idx_v.at[pl.ds(i * CHUNK, CHUNK)]],
                    bufs[i], sgs[i])

        for i in range(MAXC):
            b = i % NBUF

            @pl.when(chunk_id(i) < NCHUNKS)
            def _(i=i, b=b):
                pltpu.make_async_copy(
                    x_hbm.at[idx_v.at[pl.ds(i * CHUNK, CHUNK)]], bufs[b], sgs[b]).wait()
                pltpu.async_copy(bufs[b], out_slab(i), sss[b])

            if i + 2 < MAXC:
                b2 = (i + 2) % NBUF

                @pl.when(chunk_id(i + 2) < NCHUNKS)
                def _(i=i, b2=b2):
                    if i >= 2:
                        pltpu.make_async_copy(
                            bufs[b2], out_slab(i - 2), sss[b2]).wait()
                    pltpu.async_copy(
                        x_hbm.at[idx_v.at[pl.ds((i + 2) * CHUNK, CHUNK)]],
                        bufs[b2], sgs[b2])

        for i in range(MAXC):
            cond = chunk_id(i) < NCHUNKS
            if i + 4 < MAXC:
                cond &= chunk_id(i + 4) >= NCHUNKS

            @pl.when(cond)
            def _(i=i):
                pltpu.make_async_copy(
                    bufs[i % NBUF], out_slab(i), sss[i % NBUF]).wait()

    return body


def kernel(x: jnp.ndarray) -> jnp.ndarray:
    xflat = x.transpose(1, 0, 2).reshape(T * B, D)
    idx = jnp.asarray(_gather_indices())
    out = _sc_gather()(xflat, idx)
    return out.reshape(K + 1, B, D).transpose(1, 0, 2)


# speedup vs baseline: 5.9754x; 1.0003x over previous
import functools

import jax
import jax.numpy as jnp
import numpy as np
from jax import lax
from jax.experimental import pallas as pl
from jax.experimental.pallas import tpu as pltpu
from jax.experimental.pallas import tpu_sc as plsc

B = 64
T = 577
N = T - 1
K = N // 2
D = 768
R = B * (K + 1)
CHUNK = 32
NCHUNKS = R // CHUNK
NW = 32
MAXC = -(-NCHUNKS // NW)
NBUF = 4
IDXW = -(-MAXC * CHUNK // 128) * 128


@functools.cache
def _gather_indices() -> np.ndarray:
    with jax.ensure_compile_time_eval():
        rand = jax.random.normal(jax.random.key(1), (B, N), dtype=jnp.float32)
        _, keep = lax.top_k(rand, K)
        tok = np.concatenate(
            [np.zeros((B, 1), np.int32), 1 + np.asarray(keep, np.int32)],
            axis=1)
    b = np.arange(B, dtype=np.int32)[None, :]
    q = tok.T * B + b
    q = q.reshape(NCHUNKS, CHUNK)
    pad = np.zeros((NW * MAXC - NCHUNKS, CHUNK), np.int32)
    q = np.concatenate([q, pad], axis=0)
    q = q.reshape(MAXC, NW, CHUNK).transpose(1, 0, 2).reshape(NW, MAXC * CHUNK)
    out = np.zeros((NW, IDXW), np.int32)
    out[:, :MAXC * CHUNK] = q
    return out


@functools.cache
def _sc_gather():
    @functools.partial(
        pl.kernel,
        out_type=jax.ShapeDtypeStruct((R, D), jnp.float32),
        mesh=plsc.VectorSubcoreMesh(core_axis_name="c", subcore_axis_name="s"),
        compiler_params=pltpu.CompilerParams(use_tc_tiling_on_sc=True),
        scratch_types=[
            pltpu.VMEM((IDXW,), jnp.int32),
            [pltpu.VMEM((CHUNK, D), jnp.float32) for _ in range(NBUF)],
            [pltpu.SemaphoreType.DMA for _ in range(NBUF)],
            [pltpu.SemaphoreType.DMA for _ in range(NBUF)],
        ],
    )
    def body(x_hbm, idx_hbm, out_hbm, idx_v, bufs, sgs, sss):
        wid = lax.axis_index("s") * 2 + lax.axis_index("c")

        pltpu.sync_copy(idx_hbm.at[wid], idx_v)

        def chunk_id(i):
            return wid + i * NW

        def out_slab(i):
            return out_hbm.at[pl.ds(chunk_id(i) * CHUNK, CHUNK)]

        for i in range(2):
            @pl.when(chunk_id(i) < NCHUNKS)
            def _(i=i):
                pltpu.async_copy(
                    x_hbm.at[idx_v.at[pl.ds(i * CHUNK, CHUNK)]],
                    bufs[i], sgs[i])

        for i in range(MAXC):
            b = i % NBUF

            @pl.when(chunk_id(i) < NCHUNKS)
            def _(i=i, b=b):
                pltpu.make_async_copy(
                    x_hbm.at[idx_v.at[pl.ds(i * CHUNK, CHUNK)]], bufs[b], sgs[b]).wait()
                pltpu.async_copy(bufs[b], out_slab(i), sss[b])

            if i + 2 < MAXC:
                b2 = (i + 2) % NBUF

                @pl.when(chunk_id(i + 2) < NCHUNKS)
                def _(i=i, b2=b2):
                    if i >= 2:
                        pltpu.make_async_copy(
                            bufs[b2], out_slab(i - 2), sss[b2]).wait()
                    pltpu.async_copy(
                        x_hbm.at[idx_v.at[pl.ds((i + 2) * CHUNK, CHUNK)]],
                        bufs[b2], sgs[b2])

        for i in range(MAXC):
            cond = chunk_id(i) < NCHUNKS
            if i + 4 < MAXC:
                cond &= chunk_id(i + 4) >= NCHUNKS

            @pl.when(cond)
            def _(i=i):
                pltpu.make_async_copy(
                    bufs[i % NBUF], out_slab(i), sss[i % NBUF]).wait()

    return body


def kernel(x: jnp.ndarray) -> jnp.ndarray:
    xflat = x.transpose(1, 0, 2).reshape(T * B, D)
    idx = jnp.asarray(_gather_indices())
    out = _sc_gather()(xflat, idx)
    return out.reshape(K + 1, B, D).transpose(1, 0, 2)
